# baseline (speedup 1.0000x reference)
import functools
import math

import jax
import jax.numpy as jnp
from jax import lax
from jax.experimental import pallas as pl
from jax.experimental.pallas import tpu as pltpu
from jax.experimental.pallas import tpu_sc as plsc

_VOCAB = 1000000
_D = 64
_B = 4096
_H = 200
_TOT = _B * _H
_CH = 128
_NCHUNK = _TOT // _CH
_NW = 32
_CPW = _NCHUNK // _NW
_SCALE = math.sqrt(_D)


def _sc_embed(xf, lut):
    mesh = plsc.VectorSubcoreMesh(core_axis_name="c", subcore_axis_name="s")

    @functools.partial(
        pl.kernel,
        mesh=mesh,
        out_type=jax.ShapeDtypeStruct((_TOT, _D), jnp.float32),
        scratch_types=[
            pltpu.VMEM((_CPW, _CH), jnp.int32),
            pltpu.VMEM((_CH, _D), jnp.float32),
            pltpu.SemaphoreType.DMA,
        ],
        compiler_params=pltpu.CompilerParams(use_tc_tiling_on_sc=False),
    )
    def k(x_hbm, lut_hbm, out_hbm, idx_v, buf, gsem):
        wid = lax.axis_index("s") * 2 + lax.axis_index("c")
        base = wid * _CPW
        pltpu.sync_copy(x_hbm.at[pl.ds(base, _CPW)], idx_v)

        def chunk(j, carry):
            pltpu.async_copy(lut_hbm.at[idx_v.at[j]], buf, gsem).wait()

            def srow(r, c2):
                for c in range(_D // 16):
                    sl = pl.ds(c * 16, 16)
                    buf[r, sl] = buf[r, sl] * _SCALE
                return c2

            lax.fori_loop(0, _CH, srow, 0, unroll=2)
            pltpu.sync_copy(buf, out_hbm.at[pl.ds((base + j) * _CH, _CH)])
            return carry

        lax.fori_loop(0, _CPW, chunk, 0)

    return k(xf, lut)


def kernel(x, lut):
    xf = x.reshape(_NCHUNK, _CH)
    out = _sc_embed(xf, lut)
    return out.reshape(_B, _H, _D)


# speedup vs baseline: 1.1644x; 1.1644x over previous
import functools
import math

import jax
import jax.numpy as jnp
from jax import lax
from jax.experimental import pallas as pl
from jax.experimental.pallas import tpu as pltpu
from jax.experimental.pallas import tpu_sc as plsc

_VOCAB = 1000000
_D = 64
_B = 4096
_H = 200
_TOT = _B * _H
_CH = 128
_NCHUNK = _TOT // _CH
_NW = 32
_CPW = _NCHUNK // _NW
_SCALE = math.sqrt(_D)
_NBUF = 4
_LOOK = 2


def _sc_embed(xf, lut):
    mesh = plsc.VectorSubcoreMesh(core_axis_name="c", subcore_axis_name="s")

    @functools.partial(
        pl.kernel,
        mesh=mesh,
        out_type=jax.ShapeDtypeStruct((_TOT, _D), jnp.float32),
        scratch_types=[
            pltpu.VMEM((_CPW, _CH), jnp.int32),
            [pltpu.VMEM((_CH, _D), jnp.float32) for _ in range(_NBUF)],
            [pltpu.SemaphoreType.DMA for _ in range(_NBUF)],
            [pltpu.SemaphoreType.DMA for _ in range(_NBUF)],
        ],
        compiler_params=pltpu.CompilerParams(use_tc_tiling_on_sc=False),
    )
    def k(x_hbm, lut_hbm, out_hbm, idx_v, bufs, gsems, ssems):
        wid = lax.axis_index("s") * 2 + lax.axis_index("c")
        base = wid * _CPW
        pltpu.sync_copy(x_hbm.at[pl.ds(base, _CPW)], idx_v)

        for b in range(_LOOK):
            pltpu.async_copy(lut_hbm.at[idx_v.at[b]], bufs[b], gsems[b])

        def tick(i, carry):
            j0 = i * _NBUF
            for b in range(_NBUF):
                j = j0 + b
                bg = (b + _LOOK) % _NBUF
                jg = j + _LOOK

                @pl.when(jg >= _NBUF)
                def _():
                    pltpu.make_async_copy(bufs[bg], out_hbm.at[pl.ds(0, _CH)],
                                          ssems[bg]).wait()

                @pl.when(jg < _CPW)
                def _():
                    pltpu.async_copy(lut_hbm.at[idx_v.at[jg]], bufs[bg],
                                     gsems[bg])

                pltpu.make_async_copy(lut_hbm.at[idx_v.at[j]], bufs[b],
                                      gsems[b]).wait()

                def srow(r, c2):
                    for c in range(_D // 16):
                        sl = pl.ds(c * 16, 16)
                        bufs[b][r, sl] = bufs[b][r, sl] * _SCALE
                    return c2

                lax.fori_loop(0, _CH, srow, 0, unroll=4)
                pltpu.async_copy(bufs[b],
                                 out_hbm.at[pl.ds((base + j) * _CH, _CH)],
                                 ssems[b])
            return carry

        lax.fori_loop(0, _CPW // _NBUF, tick, 0)

        for j in range(_CPW - (_NBUF - _LOOK), _CPW):
            b = j % _NBUF
            pltpu.make_async_copy(bufs[b], out_hbm.at[pl.ds(0, _CH)],
                                  ssems[b]).wait()

    return k(xf, lut)


def kernel(x, lut):
    xf = x.reshape(_NCHUNK, _CH)
    out = _sc_embed(xf, lut)
    return out.reshape(_B, _H, _D)
